# Initial kernel scaffold; baseline (speedup 1.0000x reference)
import jax
import jax.numpy as jnp
from jax.experimental import pallas as pl


def kernel(vertices, local_features, global_features, params, edge_index):
    raise NotImplementedError("write your pallas kernel here")


# speedup vs baseline: 25.2358x; 25.2358x over previous
import functools

import jax
import jax.numpy as jnp
from jax import lax
from jax.experimental import pallas as pl
from jax.experimental.pallas import tpu as pltpu
from jax.experimental.pallas import tpu_sc as plsc

N = 10000
B = 2
E = 160000
R = B * N
RB = 1000
D = 256
DH = 128
NSUB = 16
NCORE = 2
EDGES_PER_SUB = E // NSUB
CHUNK = 80
ZROWS = 125


def _sc_agg_body(nunits, d, sup, src_hbm, dst_hbm, out,
                 src_v, dst_v, idx_v, rows_v, zero_v, acc, sem):
    c = lax.axis_index("c")
    s = lax.axis_index("s")

    nlanes = d // 16
    zero16 = jnp.zeros((16,), jnp.float32)

    def zfill(r, carry):
        for j in range(nlanes):
            zero_v[r, pl.ds(j * 16, 16)] = zero16
        return carry

    lax.fori_loop(0, ZROWS, zfill, 0)

    rows_per_sub = N // NSUB
    r0 = s * rows_per_sub
    e0 = s * EDGES_PER_SUB
    nchunks = EDGES_PER_SUB // CHUNK

    for u in range(nunits):
        base = (c * nunits + u) * N

        for z in range(rows_per_sub // ZROWS):
            pltpu.sync_copy(zero_v, acc.at[pl.ds(r0 + z * ZROWS, ZROWS)])
        plsc.subcore_barrier()

        def chunk(i, carry):
            eb = e0 + i * CHUNK
            pltpu.sync_copy(src_hbm.at[pl.ds(eb, CHUNK)], src_v)
            pltpu.sync_copy(dst_hbm.at[pl.ds(eb, CHUNK)], dst_v)
            for j in range(CHUNK // 16):
                idx_v[pl.ds(j * 16, 16)] = src_v[pl.ds(j * 16, 16)] + base
            pltpu.async_copy(sup.at[idx_v], rows_v, sem).wait()
            pltpu.sync_copy(rows_v, acc.at[dst_v], add=True)
            return carry

        lax.fori_loop(0, nchunks, chunk, 0)
        plsc.subcore_barrier()

        for z in range(rows_per_sub // ZROWS):
            pltpu.sync_copy(acc.at[pl.ds(r0 + z * ZROWS, ZROWS)],
                            out.at[pl.ds(base + r0 + z * ZROWS, ZROWS)])
        plsc.subcore_barrier()


@functools.partial(jax.jit, static_argnums=(3, 4))
def _sc_agg(sup, src, dst, nunits, d):
    mesh = plsc.VectorSubcoreMesh(core_axis_name="c", subcore_axis_name="s")
    body = functools.partial(_sc_agg_body, nunits, d)
    return pl.kernel(
        body,
        out_type=jax.ShapeDtypeStruct((NCORE * nunits * N, d), jnp.float32),
        mesh=mesh,
        compiler_params=pltpu.CompilerParams(use_tc_tiling_on_sc=False),
        scratch_types=[
            pltpu.VMEM((CHUNK,), jnp.int32),
            pltpu.VMEM((CHUNK,), jnp.int32),
            pltpu.VMEM((CHUNK,), jnp.int32),
            pltpu.VMEM((CHUNK, d), jnp.float32),
            pltpu.VMEM((ZROWS, d), jnp.float32),
            pltpu.VMEM_SHARED((N, d), jnp.float32),
            pltpu.SemaphoreType.DMA,
        ],
    )(sup, src, dst)


def _half_specs(i_fixed=None):
    return [
        pl.BlockSpec((1, RB, DH), lambda i: (0, i, 0)),
        pl.BlockSpec((1, RB, DH), lambda i: (1, i, 0)),
    ]


def _cat(hl_ref, hr_ref):
    return jnp.concatenate([hl_ref[0], hr_ref[0]], axis=1)


def _split_store(out_ref, val):
    out_ref[0] = val[:, :DH]
    out_ref[1] = val[:, DH:]


def _mm_dual_body(hl, hr, wl, ws, b, t_out, u_out):
    a = _cat(hl, hr)
    _split_store(t_out, jnp.dot(a, wl[...], preferred_element_type=jnp.float32))
    _split_store(u_out, jnp.dot(a, ws[...], preferred_element_type=jnp.float32)
                 + b[...])


def _mm_dual(h, wl, ws, b):
    return pl.pallas_call(
        _mm_dual_body,
        grid=(R // RB,),
        in_specs=_half_specs() + [
            pl.BlockSpec((D, D), lambda i: (0, 0)),
            pl.BlockSpec((D, D), lambda i: (0, 0)),
            pl.BlockSpec((1, D), lambda i: (0, 0)),
        ],
        out_specs=[
            pl.BlockSpec((2, RB, DH), lambda i: (0, i, 0)),
            pl.BlockSpec((2, RB, DH), lambda i: (0, i, 0)),
        ],
        out_shape=[
            jax.ShapeDtypeStruct((2, R, DH), jnp.float32),
            jax.ShapeDtypeStruct((2, R, DH), jnp.float32),
        ],
    )(h, h, wl, ws, b.reshape(1, D))


def _mm_single_body(relu, res, hl, hr, ws, b, *rest):
    if res:
        res_ref, out = rest
    else:
        (out,) = rest
    a = _cat(hl, hr)
    u = jnp.dot(a, ws[...], preferred_element_type=jnp.float32) + b[...]
    if relu:
        u = jnp.maximum(u, 0.0)
    if res:
        u = u + jnp.concatenate([res_ref[0], res_ref[1]], axis=1)
    _split_store(out, u)


def _mm_single(h, ws, b, relu=False, res=None):
    in_specs = _half_specs() + [
        pl.BlockSpec((D, D), lambda i: (0, 0)),
        pl.BlockSpec((1, D), lambda i: (0, 0)),
    ]
    args = [h, h, ws, b.reshape(1, D)]
    if res is not None:
        in_specs.append(pl.BlockSpec((2, RB, DH), lambda i: (0, i, 0)))
        args.append(res)
    return pl.pallas_call(
        functools.partial(_mm_single_body, relu, res is not None),
        grid=(R // RB,),
        in_specs=in_specs,
        out_specs=pl.BlockSpec((2, RB, DH), lambda i: (0, i, 0)),
        out_shape=jax.ShapeDtypeStruct((2, R, DH), jnp.float32),
    )(*args)


def _mm_first_body(blocks_per_batch, av, af, g, wvl, wfl, wgl, wvs, wfs, wgs,
                   b, t_out, u_out):
    a3 = av[...]
    afull = af[...]
    gv = g[...]
    b_idx = pl.program_id(0) // blocks_per_batch

    def pick(m):
        return jnp.where(b_idx == 0, m[0:1, :], m[1:2, :])

    gl = pick(jnp.dot(gv, wgl[...], preferred_element_type=jnp.float32))
    gs = pick(jnp.dot(gv, wgs[...], preferred_element_type=jnp.float32))
    t = (jnp.dot(afull, wfl[...], preferred_element_type=jnp.float32)
         + jnp.dot(a3, wvl[...], preferred_element_type=jnp.float32)
         + gl)
    u = (jnp.dot(afull, wfs[...], preferred_element_type=jnp.float32)
         + jnp.dot(a3, wvs[...], preferred_element_type=jnp.float32)
         + gs + b[...])
    _split_store(t_out, t)
    _split_store(u_out, u)


def _mm_first(verts, loc, glob, wl, ws, b, n_dim, d_loc):
    d_glob = glob.shape[1]
    wvl, wfl, wgl = wl[:n_dim], wl[n_dim:n_dim + d_loc], wl[n_dim + d_loc:]
    wvs, wfs, wgs = ws[:n_dim], ws[n_dim:n_dim + d_loc], ws[n_dim + d_loc:]
    pad = jnp.zeros((8 - n_dim, D), jnp.float32)
    wvl = jnp.concatenate([wvl, pad], axis=0)
    wvs = jnp.concatenate([wvs, pad], axis=0)
    blocks_per_batch = N // RB
    return pl.pallas_call(
        functools.partial(_mm_first_body, blocks_per_batch),
        grid=(R // RB,),
        in_specs=[
            pl.BlockSpec((RB, 8), lambda i: (i, 0)),
            pl.BlockSpec((RB, d_loc), lambda i: (i, 0)),
            pl.BlockSpec((B, d_glob), lambda i: (0, 0)),
            pl.BlockSpec((8, D), lambda i: (0, 0)),
            pl.BlockSpec((d_loc, D), lambda i: (0, 0)),
            pl.BlockSpec((d_glob, D), lambda i: (0, 0)),
            pl.BlockSpec((8, D), lambda i: (0, 0)),
            pl.BlockSpec((d_loc, D), lambda i: (0, 0)),
            pl.BlockSpec((d_glob, D), lambda i: (0, 0)),
            pl.BlockSpec((1, D), lambda i: (0, 0)),
        ],
        out_specs=[
            pl.BlockSpec((2, RB, DH), lambda i: (0, i, 0)),
            pl.BlockSpec((2, RB, DH), lambda i: (0, i, 0)),
        ],
        out_shape=[
            jax.ShapeDtypeStruct((2, R, DH), jnp.float32),
            jax.ShapeDtypeStruct((2, R, DH), jnp.float32),
        ],
    )(verts, loc, glob, wvl, wfl, wgl, wvs, wfs, wgs, b.reshape(1, D))


def _ew_body(relu, res, agg, u, *rest):
    if res:
        res_ref, out = rest
    else:
        (out,) = rest
    v = agg[...] + u[...]
    if relu:
        v = jnp.maximum(v, 0.0)
    if res:
        v = v + res_ref[...]
    out[...] = v


def _ew(agg, u, relu=False, res=None):
    eb = 2000
    spec = pl.BlockSpec((1, eb, DH), lambda c, i: (c, i, 0))
    args = [agg, u]
    in_specs = [spec, spec]
    if res is not None:
        in_specs.append(spec)
        args.append(res)
    return pl.pallas_call(
        functools.partial(_ew_body, relu, res is not None),
        grid=(2, R // eb),
        in_specs=in_specs,
        out_specs=spec,
        out_shape=jax.ShapeDtypeStruct((2, R, DH), jnp.float32),
    )(*args)


DLAST = 16


def _mm_last_body(dual, hl, hr, *rest):
    if dual:
        wl, ws, b, t_out, u_out = rest
    else:
        ws, b, u_out = rest
    a = _cat(hl, hr)
    if dual:
        t_out[...] = jnp.dot(a, wl[...], preferred_element_type=jnp.float32)
    u_out[...] = jnp.dot(a, ws[...], preferred_element_type=jnp.float32) + b[...]


def _mm_last(h, wl, ws, b):
    dual = wl is not None
    in_specs = _half_specs()
    args = [h, h]
    out_spec = pl.BlockSpec((RB, DLAST), lambda i: (i, 0))
    if dual:
        in_specs.append(pl.BlockSpec((D, DLAST), lambda i: (0, 0)))
        args.append(wl)
    in_specs += [
        pl.BlockSpec((D, DLAST), lambda i: (0, 0)),
        pl.BlockSpec((1, DLAST), lambda i: (0, 0)),
    ]
    args += [ws, b.reshape(1, DLAST)]
    out_specs = [out_spec, out_spec] if dual else out_spec
    out_shape = jax.ShapeDtypeStruct((R, DLAST), jnp.float32)
    return pl.pallas_call(
        functools.partial(_mm_last_body, dual),
        grid=(R // RB,),
        in_specs=in_specs,
        out_specs=out_specs,
        out_shape=[out_shape, out_shape] if dual else out_shape,
    )(*args)


def _ew_last_body(agg, u, out):
    out[...] = agg[...] + u[...]


def _ew_last(agg, u):
    eb = 2000
    spec = pl.BlockSpec((eb, DLAST), lambda i: (i, 0))
    return pl.pallas_call(
        _ew_last_body,
        grid=(R // eb,),
        in_specs=[spec, spec],
        out_specs=spec,
        out_shape=jax.ShapeDtypeStruct((R, DLAST), jnp.float32),
    )(agg, u)


def kernel(vertices, local_features, global_features, params, edge_index):
    src = edge_index[0]
    dst = edge_index[1]
    n_dim = vertices.shape[2]
    d_loc = local_features.shape[2]

    verts = jnp.pad(vertices.reshape(R, n_dim), ((0, 0), (0, 8 - n_dim)))
    loc = local_features.reshape(R, d_loc)

    def agg256(t):
        return _sc_agg(t.reshape(2 * R, DH), src, dst, B, DH).reshape(2, R, DH)

    p = params["block0"]
    t, u = _mm_first(verts, loc, global_features,
                     p["g1"]["Wl"], p["g1"]["Ws"], p["g1"]["b"], n_dim, d_loc)
    xs = None

    def bottleneck(xs, p, first_tu=None):
        if first_tu is not None:
            t, u = first_tu
            h1 = [_ew(agg256(t), u, relu=True)]
        else:
            t, u = _mm_dual(xs[0], p["g1"]["Wl"], p["g1"]["Ws"], p["g1"]["b"])
            h1 = [_ew(agg256(t), u, relu=True)]
            h1 += [_mm_single(x, p["g1"]["Ws"], p["g1"]["b"], relu=True)
                   for x in xs[1:]]
        t, u = _mm_dual(h1[0], p["g2"]["Wl"], p["g2"]["Ws"], p["g2"]["b"])
        h = [_ew(agg256(t), u, relu=True, res=h1[0])]
        h += [_mm_single(hj, p["g2"]["Ws"], p["g2"]["b"], relu=True, res=hj)
              for hj in h1[1:]]
        t, u = _mm_dual(h[0], p["g3"]["Wl"], p["g3"]["Ws"], p["g3"]["b"])
        x_out = [_ew(agg256(t), u)]
        x_out += [_mm_single(hj, p["g3"]["Ws"], p["g3"]["b"]) for hj in h[1:]]
        return x_out + h

    xs = bottleneck(None, params["block0"], first_tu=(t, u))
    xs = bottleneck(xs, params["block1"])
    xs = bottleneck(xs, params["block2"])

    pl_ = params["last"]
    wl16 = jnp.pad(pl_["Wl"], ((0, 0), (0, DLAST - 3)))
    ws16 = jnp.pad(pl_["Ws"], ((0, 0), (0, DLAST - 3)))
    b16 = jnp.pad(pl_["b"], (0, DLAST - 3))
    t, u0 = _mm_last(xs[0], wl16, ws16, b16)
    agg = _sc_agg(t, src, dst, 1, DLAST)
    outs = [_ew_last(agg, u0)]
    outs += [_mm_last(x, None, ws16, b16) for x in xs[1:]]

    stacked = jnp.concatenate([o.reshape(B, N, DLAST) for o in outs], axis=1)
    return stacked[:, :, :3]


# speedup vs baseline: 36.8936x; 1.4620x over previous
import functools

import jax
import jax.numpy as jnp
from jax import lax
from jax.experimental import pallas as pl
from jax.experimental.pallas import tpu as pltpu
from jax.experimental.pallas import tpu_sc as plsc

N = 10000
B = 2
E = 160000
R = B * N
RB = 1000
D = 256
NSP = 4
DS = D // NSP
NSUB = 16
NCORE = 2
EDGES_PER_SUB = E // NSUB
CHUNK = 80
NCHUNK = EDGES_PER_SUB // CHUNK
G = 5
NG = NCHUNK // G
RPS = N // NSUB


def _sc_agg_body(nunits, d, sup, src3, dst3, zeros_hbm, out,
                 dst_v, idx_v, rows_v, acc, sem):
    c = lax.axis_index("c")
    s = lax.axis_index("s")

    pltpu.sync_copy(src3.at[s], idx_v)
    pltpu.sync_copy(dst3.at[s], dst_v)

    r0 = s * RPS
    base0 = c * (nunits * N)

    def shift_idx(delta):
        def row(rr, carry):
            for j in range(CHUNK // 16):
                sl = pl.ds(j * 16, 16)
                idx_v[rr, sl] = idx_v[rr, sl] + delta
            return carry
        lax.fori_loop(0, NCHUNK, row, 0)

    shift_idx(base0)

    for u in range(nunits):
        if u > 0:
            shift_idx(N)
        base = base0 + u * N

        pltpu.sync_copy(zeros_hbm, acc.at[pl.ds(r0, RPS)])
        plsc.subcore_barrier()

        for j in range(G):
            pltpu.async_copy(sup.at[idx_v.at[j]], rows_v.at[j], sem)

        def group(g, carry):
            gb = g * G
            buf0 = lax.rem(g, 2) * G
            nbuf0 = lax.rem(g + 1, 2) * G
            for j in range(G):
                pltpu.make_async_copy(sup.at[pl.ds(0, CHUNK)],
                                      rows_v.at[0], sem).wait()

            @pl.when(g + 1 < NG)
            def _():
                for j in range(G):
                    pltpu.async_copy(sup.at[idx_v.at[gb + G + j]],
                                     rows_v.at[nbuf0 + j], sem)

            for j in range(G):
                pltpu.sync_copy(rows_v.at[buf0 + j],
                                acc.at[dst_v.at[gb + j]], add=True)
            return carry

        lax.fori_loop(0, NG, group, 0)
        plsc.subcore_barrier()

        pltpu.sync_copy(acc.at[pl.ds(r0, RPS)],
                        out.at[pl.ds(base + r0, RPS)])


@functools.partial(jax.jit, static_argnums=(3, 4))
def _sc_agg(sup, src3, dst3, nunits, d):
    mesh = plsc.VectorSubcoreMesh(core_axis_name="c", subcore_axis_name="s")
    body = functools.partial(_sc_agg_body, nunits, d)
    zeros_hbm = jnp.zeros((RPS, d), jnp.float32)
    return pl.kernel(
        body,
        out_type=jax.ShapeDtypeStruct((NCORE * nunits * N, d), jnp.float32),
        mesh=mesh,
        compiler_params=pltpu.CompilerParams(use_tc_tiling_on_sc=False),
        scratch_types=[
            pltpu.VMEM((NCHUNK, CHUNK), jnp.int32),
            pltpu.VMEM((NCHUNK, CHUNK), jnp.int32),
            pltpu.VMEM((2 * G, CHUNK, d), jnp.float32),
            pltpu.VMEM_SHARED((N, d), jnp.float32),
            pltpu.SemaphoreType.DMA,
        ],
    )(sup, src3, dst3, zeros_hbm)


def _part_specs():
    def mk(q):
        return pl.BlockSpec((1, RB, DS), lambda i, q=q: (q, i, 0))
    return [mk(q) for q in range(NSP)]


def _cat(refs):
    return jnp.concatenate([r[0] for r in refs], axis=1)


def _split_store(out_ref, val):
    for q in range(NSP):
        out_ref[q] = val[:, q * DS:(q + 1) * DS]


_FULL_SPEC = pl.BlockSpec((NSP, RB, DS), lambda i: (0, i, 0))


def _mm_dual_body(*refs):
    parts, (wl, ws, b, t_out, u_out) = refs[:NSP], refs[NSP:]
    a = _cat(parts)
    _split_store(t_out, jnp.dot(a, wl[...], preferred_element_type=jnp.float32))
    _split_store(u_out, jnp.dot(a, ws[...], preferred_element_type=jnp.float32)
                 + b[...])


def _mm_dual(h, wl, ws, b):
    return pl.pallas_call(
        _mm_dual_body,
        grid=(R // RB,),
        in_specs=_part_specs() + [
            pl.BlockSpec((D, D), lambda i: (0, 0)),
            pl.BlockSpec((D, D), lambda i: (0, 0)),
            pl.BlockSpec((1, D), lambda i: (0, 0)),
        ],
        out_specs=[_FULL_SPEC, _FULL_SPEC],
        out_shape=[
            jax.ShapeDtypeStruct((NSP, R, DS), jnp.float32),
            jax.ShapeDtypeStruct((NSP, R, DS), jnp.float32),
        ],
    )(*([h] * NSP), wl, ws, b.reshape(1, D))


def _mm_single_body(relu, res, *refs):
    parts = refs[:NSP]
    if res:
        ws, b, res_ref, out = refs[NSP:]
    else:
        ws, b, out = refs[NSP:]
    a = _cat(parts)
    u = jnp.dot(a, ws[...], preferred_element_type=jnp.float32) + b[...]
    if relu:
        u = jnp.maximum(u, 0.0)
    if res:
        u = u + jnp.concatenate([res_ref[q] for q in range(NSP)], axis=1)
    _split_store(out, u)


def _mm_single(h, ws, b, relu=False, res=None):
    in_specs = _part_specs() + [
        pl.BlockSpec((D, D), lambda i: (0, 0)),
        pl.BlockSpec((1, D), lambda i: (0, 0)),
    ]
    args = [h] * NSP + [ws, b.reshape(1, D)]
    if res is not None:
        in_specs.append(_FULL_SPEC)
        args.append(res)
    return pl.pallas_call(
        functools.partial(_mm_single_body, relu, res is not None),
        grid=(R // RB,),
        in_specs=in_specs,
        out_specs=_FULL_SPEC,
        out_shape=jax.ShapeDtypeStruct((NSP, R, DS), jnp.float32),
    )(*args)


def _mm_first_body(blocks_per_batch, av, af, g, wvl, wfl, wgl, wvs, wfs, wgs,
                   b, t_out, u_out):
    a3 = av[...]
    afull = af[...]
    gv = g[...]
    b_idx = pl.program_id(0) // blocks_per_batch

    def pick(m):
        return jnp.where(b_idx == 0, m[0:1, :], m[1:2, :])

    gl = pick(jnp.dot(gv, wgl[...], preferred_element_type=jnp.float32))
    gs = pick(jnp.dot(gv, wgs[...], preferred_element_type=jnp.float32))
    t = (jnp.dot(afull, wfl[...], preferred_element_type=jnp.float32)
         + jnp.dot(a3, wvl[...], preferred_element_type=jnp.float32)
         + gl)
    u = (jnp.dot(afull, wfs[...], preferred_element_type=jnp.float32)
         + jnp.dot(a3, wvs[...], preferred_element_type=jnp.float32)
         + gs + b[...])
    _split_store(t_out, t)
    _split_store(u_out, u)


def _mm_first(verts, loc, glob, wl, ws, b, n_dim, d_loc):
    d_glob = glob.shape[1]
    wvl, wfl, wgl = wl[:n_dim], wl[n_dim:n_dim + d_loc], wl[n_dim + d_loc:]
    wvs, wfs, wgs = ws[:n_dim], ws[n_dim:n_dim + d_loc], ws[n_dim + d_loc:]
    pad = jnp.zeros((8 - n_dim, D), jnp.float32)
    wvl = jnp.concatenate([wvl, pad], axis=0)
    wvs = jnp.concatenate([wvs, pad], axis=0)
    blocks_per_batch = N // RB
    return pl.pallas_call(
        functools.partial(_mm_first_body, blocks_per_batch),
        grid=(R // RB,),
        in_specs=[
            pl.BlockSpec((RB, 8), lambda i: (i, 0)),
            pl.BlockSpec((RB, d_loc), lambda i: (i, 0)),
            pl.BlockSpec((B, d_glob), lambda i: (0, 0)),
            pl.BlockSpec((8, D), lambda i: (0, 0)),
            pl.BlockSpec((d_loc, D), lambda i: (0, 0)),
            pl.BlockSpec((d_glob, D), lambda i: (0, 0)),
            pl.BlockSpec((8, D), lambda i: (0, 0)),
            pl.BlockSpec((d_loc, D), lambda i: (0, 0)),
            pl.BlockSpec((d_glob, D), lambda i: (0, 0)),
            pl.BlockSpec((1, D), lambda i: (0, 0)),
        ],
        out_specs=[_FULL_SPEC, _FULL_SPEC],
        out_shape=[
            jax.ShapeDtypeStruct((NSP, R, DS), jnp.float32),
            jax.ShapeDtypeStruct((NSP, R, DS), jnp.float32),
        ],
    )(verts, loc, glob, wvl, wfl, wgl, wvs, wfs, wgs, b.reshape(1, D))


def _ew_body(relu, res, agg, u, *rest):
    if res:
        res_ref, out = rest
    else:
        (out,) = rest
    v = agg[...] + u[...]
    if relu:
        v = jnp.maximum(v, 0.0)
    if res:
        v = v + res_ref[...]
    out[...] = v


def _ew(agg, u, relu=False, res=None):
    eb = 2000
    spec = pl.BlockSpec((1, eb, DS), lambda c, i: (c, i, 0))
    args = [agg, u]
    in_specs = [spec, spec]
    if res is not None:
        in_specs.append(spec)
        args.append(res)
    return pl.pallas_call(
        functools.partial(_ew_body, relu, res is not None),
        grid=(NSP, R // eb),
        in_specs=in_specs,
        out_specs=spec,
        out_shape=jax.ShapeDtypeStruct((NSP, R, DS), jnp.float32),
    )(*args)


DLAST = 16


def _mm_last_body(dual, *refs):
    parts = refs[:NSP]
    if dual:
        wl, ws, b, t_out, u_out = refs[NSP:]
    else:
        ws, b, u_out = refs[NSP:]
    a = _cat(parts)
    if dual:
        t_out[...] = jnp.dot(a, wl[...], preferred_element_type=jnp.float32)
    u_out[...] = jnp.dot(a, ws[...], preferred_element_type=jnp.float32) + b[...]


def _mm_last(h, wl, ws, b):
    dual = wl is not None
    in_specs = _part_specs()
    args = [h] * NSP
    out_spec = pl.BlockSpec((RB, DLAST), lambda i: (i, 0))
    if dual:
        in_specs.append(pl.BlockSpec((D, DLAST), lambda i: (0, 0)))
        args.append(wl)
    in_specs += [
        pl.BlockSpec((D, DLAST), lambda i: (0, 0)),
        pl.BlockSpec((1, DLAST), lambda i: (0, 0)),
    ]
    args += [ws, b.reshape(1, DLAST)]
    out_specs = [out_spec, out_spec] if dual else out_spec
    out_shape = jax.ShapeDtypeStruct((R, DLAST), jnp.float32)
    return pl.pallas_call(
        functools.partial(_mm_last_body, dual),
        grid=(R // RB,),
        in_specs=in_specs,
        out_specs=out_specs,
        out_shape=[out_shape, out_shape] if dual else out_shape,
    )(*args)


def _ew_last_body(agg, u, out):
    out[...] = agg[...] + u[...]


def _ew_last(agg, u):
    eb = 2000
    spec = pl.BlockSpec((eb, DLAST), lambda i: (i, 0))
    return pl.pallas_call(
        _ew_last_body,
        grid=(R // eb,),
        in_specs=[spec, spec],
        out_specs=spec,
        out_shape=jax.ShapeDtypeStruct((R, DLAST), jnp.float32),
    )(agg, u)


def kernel(vertices, local_features, global_features, params, edge_index):
    src = edge_index[0].reshape(NSUB, NCHUNK, CHUNK)
    dst = edge_index[1].reshape(NSUB, NCHUNK, CHUNK)
    n_dim = vertices.shape[2]
    d_loc = local_features.shape[2]

    verts = jnp.pad(vertices.reshape(R, n_dim), ((0, 0), (0, 8 - n_dim)))
    loc = local_features.reshape(R, d_loc)

    def agg256(t):
        return _sc_agg(t.reshape(NSP * R, DS), src, dst,
                       NSP * B // NCORE, DS).reshape(NSP, R, DS)

    p = params["block0"]
    t, u = _mm_first(verts, loc, global_features,
                     p["g1"]["Wl"], p["g1"]["Ws"], p["g1"]["b"], n_dim, d_loc)
    xs = None

    def bottleneck(xs, p, first_tu=None):
        if first_tu is not None:
            t, u = first_tu
            h1 = [_ew(agg256(t), u, relu=True)]
        else:
            t, u = _mm_dual(xs[0], p["g1"]["Wl"], p["g1"]["Ws"], p["g1"]["b"])
            h1 = [_ew(agg256(t), u, relu=True)]
            h1 += [_mm_single(x, p["g1"]["Ws"], p["g1"]["b"], relu=True)
                   for x in xs[1:]]
        t, u = _mm_dual(h1[0], p["g2"]["Wl"], p["g2"]["Ws"], p["g2"]["b"])
        h = [_ew(agg256(t), u, relu=True, res=h1[0])]
        h += [_mm_single(hj, p["g2"]["Ws"], p["g2"]["b"], relu=True, res=hj)
              for hj in h1[1:]]
        t, u = _mm_dual(h[0], p["g3"]["Wl"], p["g3"]["Ws"], p["g3"]["b"])
        x_out = [_ew(agg256(t), u)]
        x_out += [_mm_single(hj, p["g3"]["Ws"], p["g3"]["b"]) for hj in h[1:]]
        return x_out + h

    xs = bottleneck(None, params["block0"], first_tu=(t, u))
    xs = bottleneck(xs, params["block1"])
    xs = bottleneck(xs, params["block2"])

    pl_ = params["last"]
    wl16 = jnp.pad(pl_["Wl"], ((0, 0), (0, DLAST - 3)))
    ws16 = jnp.pad(pl_["Ws"], ((0, 0), (0, DLAST - 3)))
    b16 = jnp.pad(pl_["b"], (0, DLAST - 3))
    t, u0 = _mm_last(xs[0], wl16, ws16, b16)
    agg = _sc_agg(t, src, dst, 1, DLAST)
    outs = [_ew_last(agg, u0)]
    outs += [_mm_last(x, None, ws16, b16) for x in xs[1:]]

    stacked = jnp.concatenate([o.reshape(B, N, DLAST) for o in outs], axis=1)
    return stacked[:, :, :3]


# speedup vs baseline: 47.9201x; 1.2989x over previous
import functools

import jax
import jax.numpy as jnp
from jax import lax
from jax.experimental import pallas as pl
from jax.experimental.pallas import tpu as pltpu
from jax.experimental.pallas import tpu_sc as plsc

N = 10000
B = 2
E = 160000
R = B * N
RB = 1000
D = 256
NSP = 2
DS = D // NSP
DSC = 64
UNITS = 4
NSUB = 16
NCORE = 2
EDGES_PER_SUB = E // NSUB
CHUNK = 80
NCHUNK = EDGES_PER_SUB // CHUNK
G = 5
NG = NCHUNK // G
RPS = N // NSUB


def _sc_agg_body(interleave, d, sup, src3, dst3, zeros_hbm, out,
                 dst_v, idx_v, rows_v, acc, sem):
    c = lax.axis_index("c")
    s = lax.axis_index("s")

    pltpu.sync_copy(src3.at[s], idx_v)
    pltpu.sync_copy(dst3.at[s], dst_v)

    r0 = s * RPS

    def adjust_idx(delta, double=False):
        def row(rr, carry):
            for j in range(CHUNK // 16):
                sl = pl.ds(j * 16, 16)
                v = idx_v[rr, sl]
                if double:
                    v = v + v
                idx_v[rr, sl] = v + delta
            return carry
        lax.fori_loop(0, NCHUNK, row, 0)

    if interleave:
        adjust_idx(c * (2 * R), double=True)
        deltas = [None, 2 * N, 1 - 2 * N, 2 * N]
        units = UNITS
    else:
        adjust_idx(c * N)
        units = 1

    for u in range(units):
        if u > 0:
            adjust_idx(deltas[u])

        pltpu.sync_copy(zeros_hbm, acc.at[pl.ds(r0, RPS)])
        plsc.subcore_barrier()

        for j in range(G):
            pltpu.async_copy(sup.at[idx_v.at[j]], rows_v.at[j], sem)

        def group(g, carry):
            gb = g * G
            buf0 = lax.rem(g, 2) * G
            nbuf0 = lax.rem(g + 1, 2) * G
            for j in range(G):
                pltpu.make_async_copy(sup.at[pl.ds(0, CHUNK)],
                                      rows_v.at[0], sem).wait()

            @pl.when(g + 1 < NG)
            def _():
                for j in range(G):
                    pltpu.async_copy(sup.at[idx_v.at[gb + G + j]],
                                     rows_v.at[nbuf0 + j], sem)

            for j in range(G):
                pltpu.sync_copy(rows_v.at[buf0 + j],
                                acc.at[dst_v.at[gb + j]], add=True)
            return carry

        lax.fori_loop(0, NG, group, 0)
        plsc.subcore_barrier()

        if interleave:
            qq, b = u // 2, u % 2
            pltpu.sync_copy(acc.at[pl.ds(r0, RPS)],
                            out.at[pl.ds(c * R + b * N + r0, RPS), qq])
        else:
            pltpu.sync_copy(acc.at[pl.ds(r0, RPS)],
                            out.at[pl.ds(c * N + r0, RPS)])


@functools.partial(jax.jit, static_argnums=(3, 4))
def _sc_agg(sup, src3, dst3, interleave, d):
    mesh = plsc.VectorSubcoreMesh(core_axis_name="c", subcore_axis_name="s")
    body = functools.partial(_sc_agg_body, interleave, d)
    zeros_hbm = jnp.zeros((RPS, d), jnp.float32)
    out_shape = (2 * R, 2, DSC) if interleave else (R, d)
    return pl.kernel(
        body,
        out_type=jax.ShapeDtypeStruct(out_shape, jnp.float32),
        mesh=mesh,
        compiler_params=pltpu.CompilerParams(use_tc_tiling_on_sc=False),
        scratch_types=[
            pltpu.VMEM((NCHUNK, CHUNK), jnp.int32),
            pltpu.VMEM((NCHUNK, CHUNK), jnp.int32),
            pltpu.VMEM((2 * G, CHUNK, d), jnp.float32),
            pltpu.VMEM_SHARED((N, d), jnp.float32),
            pltpu.SemaphoreType.DMA,
        ],
    )(sup, src3, dst3, zeros_hbm)


def _part_specs():
    def mk(q):
        return pl.BlockSpec((1, RB, DS), lambda i, q=q: (q, i, 0))
    return [mk(q) for q in range(NSP)]


def _cat(refs):
    return jnp.concatenate([r[0] for r in refs], axis=1)


def _split_store(out_ref, val):
    for q in range(NSP):
        out_ref[q] = val[:, q * DS:(q + 1) * DS]


_FULL_SPEC = pl.BlockSpec((NSP, RB, DS), lambda i: (0, i, 0))


def _mm_dual_body(*refs):
    parts, (wl, ws, b, t_out, u_out) = refs[:NSP], refs[NSP:]
    a = _cat(parts)
    _split_store(t_out, jnp.dot(a, wl[...], preferred_element_type=jnp.float32))
    _split_store(u_out, jnp.dot(a, ws[...], preferred_element_type=jnp.float32)
                 + b[...])


def _mm_dual(h, wl, ws, b):
    return pl.pallas_call(
        _mm_dual_body,
        grid=(R // RB,),
        in_specs=_part_specs() + [
            pl.BlockSpec((D, D), lambda i: (0, 0)),
            pl.BlockSpec((D, D), lambda i: (0, 0)),
            pl.BlockSpec((1, D), lambda i: (0, 0)),
        ],
        out_specs=[_FULL_SPEC, _FULL_SPEC],
        out_shape=[
            jax.ShapeDtypeStruct((NSP, R, DS), jnp.float32),
            jax.ShapeDtypeStruct((NSP, R, DS), jnp.float32),
        ],
    )(*([h] * NSP), wl, ws, b.reshape(1, D))


def _mm_single_body(relu, res, *refs):
    parts = refs[:NSP]
    if res:
        ws, b, res_ref, out = refs[NSP:]
    else:
        ws, b, out = refs[NSP:]
    a = _cat(parts)
    u = jnp.dot(a, ws[...], preferred_element_type=jnp.float32) + b[...]
    if relu:
        u = jnp.maximum(u, 0.0)
    if res:
        u = u + jnp.concatenate([res_ref[q] for q in range(NSP)], axis=1)
    _split_store(out, u)


def _mm_single(h, ws, b, relu=False, res=None):
    in_specs = _part_specs() + [
        pl.BlockSpec((D, D), lambda i: (0, 0)),
        pl.BlockSpec((1, D), lambda i: (0, 0)),
    ]
    args = [h] * NSP + [ws, b.reshape(1, D)]
    if res is not None:
        in_specs.append(_FULL_SPEC)
        args.append(res)
    return pl.pallas_call(
        functools.partial(_mm_single_body, relu, res is not None),
        grid=(R // RB,),
        in_specs=in_specs,
        out_specs=_FULL_SPEC,
        out_shape=jax.ShapeDtypeStruct((NSP, R, DS), jnp.float32),
    )(*args)


def _mm_first_body(blocks_per_batch, av, af, g, wvl, wfl, wgl, wvs, wfs, wgs,
                   b, t_out, u_out):
    a3 = av[...]
    afull = af[...]
    gv = g[...]
    b_idx = pl.program_id(0) // blocks_per_batch

    def pick(m):
        return jnp.where(b_idx == 0, m[0:1, :], m[1:2, :])

    gl = pick(jnp.dot(gv, wgl[...], preferred_element_type=jnp.float32))
    gs = pick(jnp.dot(gv, wgs[...], preferred_element_type=jnp.float32))
    t = (jnp.dot(afull, wfl[...], preferred_element_type=jnp.float32)
         + jnp.dot(a3, wvl[...], preferred_element_type=jnp.float32)
         + gl)
    u = (jnp.dot(afull, wfs[...], preferred_element_type=jnp.float32)
         + jnp.dot(a3, wvs[...], preferred_element_type=jnp.float32)
         + gs + b[...])
    _split_store(t_out, t)
    _split_store(u_out, u)


def _mm_first(verts, loc, glob, wl, ws, b, n_dim, d_loc):
    d_glob = glob.shape[1]
    wvl, wfl, wgl = wl[:n_dim], wl[n_dim:n_dim + d_loc], wl[n_dim + d_loc:]
    wvs, wfs, wgs = ws[:n_dim], ws[n_dim:n_dim + d_loc], ws[n_dim + d_loc:]
    pad = jnp.zeros((8 - n_dim, D), jnp.float32)
    wvl = jnp.concatenate([wvl, pad], axis=0)
    wvs = jnp.concatenate([wvs, pad], axis=0)
    blocks_per_batch = N // RB
    return pl.pallas_call(
        functools.partial(_mm_first_body, blocks_per_batch),
        grid=(R // RB,),
        in_specs=[
            pl.BlockSpec((RB, 8), lambda i: (i, 0)),
            pl.BlockSpec((RB, d_loc), lambda i: (i, 0)),
            pl.BlockSpec((B, d_glob), lambda i: (0, 0)),
            pl.BlockSpec((8, D), lambda i: (0, 0)),
            pl.BlockSpec((d_loc, D), lambda i: (0, 0)),
            pl.BlockSpec((d_glob, D), lambda i: (0, 0)),
            pl.BlockSpec((8, D), lambda i: (0, 0)),
            pl.BlockSpec((d_loc, D), lambda i: (0, 0)),
            pl.BlockSpec((d_glob, D), lambda i: (0, 0)),
            pl.BlockSpec((1, D), lambda i: (0, 0)),
        ],
        out_specs=[_FULL_SPEC, _FULL_SPEC],
        out_shape=[
            jax.ShapeDtypeStruct((NSP, R, DS), jnp.float32),
            jax.ShapeDtypeStruct((NSP, R, DS), jnp.float32),
        ],
    )(verts, loc, glob, wvl, wfl, wgl, wvs, wfs, wgs, b.reshape(1, D))


def _ew_body(relu, res, agg, u, *rest):
    if res:
        res_ref, out = rest
    else:
        (out,) = rest
    v = agg[...] + u[...]
    if relu:
        v = jnp.maximum(v, 0.0)
    if res:
        v = v + res_ref[...]
    out[...] = v


def _ew(agg, u, relu=False, res=None):
    eb = 2000
    spec = pl.BlockSpec((1, eb, DS), lambda c, i: (c, i, 0))
    args = [agg, u]
    in_specs = [spec, spec]
    if res is not None:
        in_specs.append(spec)
        args.append(res)
    return pl.pallas_call(
        functools.partial(_ew_body, relu, res is not None),
        grid=(NSP, R // eb),
        in_specs=in_specs,
        out_specs=spec,
        out_shape=jax.ShapeDtypeStruct((NSP, R, DS), jnp.float32),
    )(*args)


DLAST = 16


def _mm_last_body(dual, *refs):
    parts = refs[:NSP]
    if dual:
        wl, ws, b, t_out, u_out = refs[NSP:]
    else:
        ws, b, u_out = refs[NSP:]
    a = _cat(parts)
    if dual:
        t_out[...] = jnp.dot(a, wl[...], preferred_element_type=jnp.float32)
    u_out[...] = jnp.dot(a, ws[...], preferred_element_type=jnp.float32) + b[...]


def _mm_last(h, wl, ws, b):
    dual = wl is not None
    in_specs = _part_specs()
    args = [h] * NSP
    out_spec = pl.BlockSpec((RB, DLAST), lambda i: (i, 0))
    if dual:
        in_specs.append(pl.BlockSpec((D, DLAST), lambda i: (0, 0)))
        args.append(wl)
    in_specs += [
        pl.BlockSpec((D, DLAST), lambda i: (0, 0)),
        pl.BlockSpec((1, DLAST), lambda i: (0, 0)),
    ]
    args += [ws, b.reshape(1, DLAST)]
    out_specs = [out_spec, out_spec] if dual else out_spec
    out_shape = jax.ShapeDtypeStruct((R, DLAST), jnp.float32)
    return pl.pallas_call(
        functools.partial(_mm_last_body, dual),
        grid=(R // RB,),
        in_specs=in_specs,
        out_specs=out_specs,
        out_shape=[out_shape, out_shape] if dual else out_shape,
    )(*args)


def _ew_last_body(agg, u, out):
    out[...] = agg[...] + u[...]


def _ew_last(agg, u):
    eb = 2000
    spec = pl.BlockSpec((eb, DLAST), lambda i: (i, 0))
    return pl.pallas_call(
        _ew_last_body,
        grid=(R // eb,),
        in_specs=[spec, spec],
        out_specs=spec,
        out_shape=jax.ShapeDtypeStruct((R, DLAST), jnp.float32),
    )(agg, u)


def kernel(vertices, local_features, global_features, params, edge_index):
    src = edge_index[0].reshape(NSUB, NCHUNK, CHUNK)
    dst = edge_index[1].reshape(NSUB, NCHUNK, CHUNK)
    n_dim = vertices.shape[2]
    d_loc = local_features.shape[2]

    verts = jnp.pad(vertices.reshape(R, n_dim), ((0, 0), (0, 8 - n_dim)))
    loc = local_features.reshape(R, d_loc)

    def agg256(t):
        return _sc_agg(t.reshape(2 * R * 2, DSC), src, dst,
                       True, DSC).reshape(NSP, R, DS)

    p = params["block0"]
    t, u = _mm_first(verts, loc, global_features,
                     p["g1"]["Wl"], p["g1"]["Ws"], p["g1"]["b"], n_dim, d_loc)
    xs = None

    def bottleneck(xs, p, first_tu=None):
        if first_tu is not None:
            t, u = first_tu
            h1 = [_ew(agg256(t), u, relu=True)]
        else:
            t, u = _mm_dual(xs[0], p["g1"]["Wl"], p["g1"]["Ws"], p["g1"]["b"])
            h1 = [_ew(agg256(t), u, relu=True)]
            h1 += [_mm_single(x, p["g1"]["Ws"], p["g1"]["b"], relu=True)
                   for x in xs[1:]]
        t, u = _mm_dual(h1[0], p["g2"]["Wl"], p["g2"]["Ws"], p["g2"]["b"])
        h = [_ew(agg256(t), u, relu=True, res=h1[0])]
        h += [_mm_single(hj, p["g2"]["Ws"], p["g2"]["b"], relu=True, res=hj)
              for hj in h1[1:]]
        t, u = _mm_dual(h[0], p["g3"]["Wl"], p["g3"]["Ws"], p["g3"]["b"])
        x_out = [_ew(agg256(t), u)]
        x_out += [_mm_single(hj, p["g3"]["Ws"], p["g3"]["b"]) for hj in h[1:]]
        return x_out + h

    xs = bottleneck(None, params["block0"], first_tu=(t, u))
    xs = bottleneck(xs, params["block1"])
    xs = bottleneck(xs, params["block2"])

    pl_ = params["last"]
    wl16 = jnp.pad(pl_["Wl"], ((0, 0), (0, DLAST - 3)))
    ws16 = jnp.pad(pl_["Ws"], ((0, 0), (0, DLAST - 3)))
    b16 = jnp.pad(pl_["b"], (0, DLAST - 3))
    t, u0 = _mm_last(xs[0], wl16, ws16, b16)
    agg = _sc_agg(t, src, dst, False, DLAST)
    outs = [_ew_last(agg, u0)]
    outs += [_mm_last(x, None, ws16, b16) for x in xs[1:]]

    stacked = jnp.concatenate([o.reshape(B, N, DLAST) for o in outs], axis=1)
    return stacked[:, :, :3]


# speedup vs baseline: 51.2667x; 1.0698x over previous
import functools

import jax
import jax.numpy as jnp
from jax import lax
from jax.experimental import pallas as pl
from jax.experimental.pallas import tpu as pltpu
from jax.experimental.pallas import tpu_sc as plsc

N = 10000
B = 2
E = 160000
R = B * N
RB = 1000
D = 256
NSP = 2
DS = D // NSP
DSC = 64
UNITS = 4
NSUB = 16
NCORE = 2
EDGES_PER_SUB = E // NSUB
CHUNK = 80
NCHUNK = EDGES_PER_SUB // CHUNK
G = 5
NG = NCHUNK // G
RPS = N // NSUB


def _sc_agg_body(interleave, d, sup, src3, dst3, zeros_hbm, out,
                 dst_v, idx_v, rows_v, acc, sem):
    c = lax.axis_index("c")
    s = lax.axis_index("s")

    pltpu.sync_copy(src3.at[s], idx_v)
    pltpu.sync_copy(dst3.at[s], dst_v)

    r0 = s * RPS

    def adjust_idx(delta, double=False):
        def row(rr, carry):
            for j in range(CHUNK // 16):
                sl = pl.ds(j * 16, 16)
                v = idx_v[rr, sl]
                if double:
                    v = v + v
                idx_v[rr, sl] = v + delta
            return carry
        lax.fori_loop(0, NCHUNK, row, 0)

    if interleave:
        adjust_idx(c * (2 * R), double=True)
        deltas = [None, 2 * N, 1 - 2 * N, 2 * N]
        units = UNITS
    else:
        adjust_idx(c * N)
        units = 1

    for u in range(units):
        if u > 0:
            adjust_idx(deltas[u])

        pltpu.sync_copy(zeros_hbm, acc.at[pl.ds(r0, RPS)])
        plsc.subcore_barrier()

        for j in range(G):
            pltpu.async_copy(sup.at[idx_v.at[j]], rows_v.at[j], sem)

        def group(g, carry):
            gb = g * G
            buf0 = lax.rem(g, 2) * G
            nbuf0 = lax.rem(g + 1, 2) * G
            for j in range(G):
                pltpu.make_async_copy(sup.at[pl.ds(0, CHUNK)],
                                      rows_v.at[0], sem).wait()

            @pl.when(g + 1 < NG)
            def _():
                for j in range(G):
                    pltpu.async_copy(sup.at[idx_v.at[gb + G + j]],
                                     rows_v.at[nbuf0 + j], sem)

            for j in range(G):
                pltpu.sync_copy(rows_v.at[buf0 + j],
                                acc.at[dst_v.at[gb + j]], add=True)
            return carry

        lax.fori_loop(0, NG, group, 0)
        plsc.subcore_barrier()

        if interleave:
            qq, b = u // 2, u % 2
            pltpu.sync_copy(acc.at[pl.ds(r0, RPS)],
                            out.at[pl.ds(c * R + b * N + r0, RPS), qq])
        else:
            pltpu.sync_copy(acc.at[pl.ds(r0, RPS)],
                            out.at[pl.ds(c * N + r0, RPS)])


@functools.partial(jax.jit, static_argnums=(3, 4))
def _sc_agg(sup, src3, dst3, interleave, d):
    mesh = plsc.VectorSubcoreMesh(core_axis_name="c", subcore_axis_name="s")
    body = functools.partial(_sc_agg_body, interleave, d)
    zeros_hbm = jnp.zeros((RPS, d), jnp.float32)
    out_shape = (2 * R, 2, DSC) if interleave else (R, d)
    return pl.kernel(
        body,
        out_type=jax.ShapeDtypeStruct(out_shape, jnp.float32),
        mesh=mesh,
        compiler_params=pltpu.CompilerParams(use_tc_tiling_on_sc=False),
        scratch_types=[
            pltpu.VMEM((NCHUNK, CHUNK), jnp.int32),
            pltpu.VMEM((NCHUNK, CHUNK), jnp.int32),
            pltpu.VMEM((2 * G, CHUNK, d), jnp.float32),
            pltpu.VMEM_SHARED((N, d), jnp.float32),
            pltpu.SemaphoreType.DMA,
        ],
    )(sup, src3, dst3, zeros_hbm)


def _part_specs():
    def mk(q):
        return pl.BlockSpec((1, RB, DS), lambda i, q=q: (q, i, 0))
    return [mk(q) for q in range(NSP)]


def _cat(refs):
    return jnp.concatenate([r[0] for r in refs], axis=1)


def _split_store(out_ref, val):
    for q in range(NSP):
        out_ref[q] = val[:, q * DS:(q + 1) * DS]


_FULL_SPEC = pl.BlockSpec((NSP, RB, DS), lambda i: (0, i, 0))


def _mm_dual_body(*refs):
    parts, (wl, ws, b, t_out, u_out) = refs[:NSP], refs[NSP:]
    a = _cat(parts)
    _split_store(t_out, jnp.dot(a, wl[...], preferred_element_type=jnp.float32))
    _split_store(u_out, jnp.dot(a, ws[...], preferred_element_type=jnp.float32)
                 + b[...])


def _mm_dual(h, wl, ws, b):
    return pl.pallas_call(
        _mm_dual_body,
        grid=(R // RB,),
        in_specs=_part_specs() + [
            pl.BlockSpec((D, D), lambda i: (0, 0)),
            pl.BlockSpec((D, D), lambda i: (0, 0)),
            pl.BlockSpec((1, D), lambda i: (0, 0)),
        ],
        out_specs=[_FULL_SPEC, _FULL_SPEC],
        out_shape=[
            jax.ShapeDtypeStruct((NSP, R, DS), jnp.float32),
            jax.ShapeDtypeStruct((NSP, R, DS), jnp.float32),
        ],
    )(*([h] * NSP), wl, ws, b.reshape(1, D))


def _mm_single_body(relu, res, *refs):
    parts = refs[:NSP]
    if res:
        ws, b, res_ref, out = refs[NSP:]
    else:
        ws, b, out = refs[NSP:]
    a = _cat(parts)
    u = jnp.dot(a, ws[...], preferred_element_type=jnp.float32) + b[...]
    if relu:
        u = jnp.maximum(u, 0.0)
    if res:
        u = u + jnp.concatenate([res_ref[q] for q in range(NSP)], axis=1)
    _split_store(out, u)


def _mm_single(h, ws, b, relu=False, res=None):
    in_specs = _part_specs() + [
        pl.BlockSpec((D, D), lambda i: (0, 0)),
        pl.BlockSpec((1, D), lambda i: (0, 0)),
    ]
    args = [h] * NSP + [ws, b.reshape(1, D)]
    if res is not None:
        in_specs.append(_FULL_SPEC)
        args.append(res)
    return pl.pallas_call(
        functools.partial(_mm_single_body, relu, res is not None),
        grid=(R // RB,),
        in_specs=in_specs,
        out_specs=_FULL_SPEC,
        out_shape=jax.ShapeDtypeStruct((NSP, R, DS), jnp.float32),
    )(*args)


def _mm_first_body(blocks_per_batch, n_dim, av, af, g, wvl, wfl, wgl,
                   wvs, wfs, wgs, b, t_out, u_out):
    a3 = av[0]
    afull = af[0]
    gv = g[...]
    b_idx = pl.program_id(0) // blocks_per_batch

    def pick(m):
        return jnp.where(b_idx == 0, m[0:1, :], m[1:2, :])

    def vterm(w):
        acc = a3[:, 0:1] * w[0:1, :]
        for k in range(1, n_dim):
            acc += a3[:, k:k + 1] * w[k:k + 1, :]
        return acc

    gl = pick(jnp.dot(gv, wgl[...], preferred_element_type=jnp.float32))
    gs = pick(jnp.dot(gv, wgs[...], preferred_element_type=jnp.float32))
    t = (jnp.dot(afull, wfl[...], preferred_element_type=jnp.float32)
         + vterm(wvl) + gl)
    u = (jnp.dot(afull, wfs[...], preferred_element_type=jnp.float32)
         + vterm(wvs) + gs + b[...])
    _split_store(t_out, t)
    _split_store(u_out, u)


def _mm_first(verts, loc, glob, wl, ws, b, n_dim, d_loc):
    d_glob = glob.shape[1]
    wvl, wfl, wgl = wl[:n_dim], wl[n_dim:n_dim + d_loc], wl[n_dim + d_loc:]
    wvs, wfs, wgs = ws[:n_dim], ws[n_dim:n_dim + d_loc], ws[n_dim + d_loc:]
    bpb = N // RB

    def row3(i):
        return (i // bpb, i % bpb, 0)

    return pl.pallas_call(
        functools.partial(_mm_first_body, bpb, n_dim),
        grid=(R // RB,),
        in_specs=[
            pl.BlockSpec((1, RB, n_dim), row3),
            pl.BlockSpec((1, RB, d_loc), row3),
            pl.BlockSpec((B, d_glob), lambda i: (0, 0)),
            pl.BlockSpec((n_dim, D), lambda i: (0, 0)),
            pl.BlockSpec((d_loc, D), lambda i: (0, 0)),
            pl.BlockSpec((d_glob, D), lambda i: (0, 0)),
            pl.BlockSpec((n_dim, D), lambda i: (0, 0)),
            pl.BlockSpec((d_loc, D), lambda i: (0, 0)),
            pl.BlockSpec((d_glob, D), lambda i: (0, 0)),
            pl.BlockSpec((1, D), lambda i: (0, 0)),
        ],
        out_specs=[_FULL_SPEC, _FULL_SPEC],
        out_shape=[
            jax.ShapeDtypeStruct((NSP, R, DS), jnp.float32),
            jax.ShapeDtypeStruct((NSP, R, DS), jnp.float32),
        ],
    )(verts, loc, glob, wvl, wfl, wgl, wvs, wfs, wgs, b.reshape(1, D))


def _ew_body(relu, res, agg, u, *rest):
    if res:
        res_ref, out = rest
    else:
        (out,) = rest
    v = agg[...] + u[...]
    if relu:
        v = jnp.maximum(v, 0.0)
    if res:
        v = v + res_ref[...]
    out[...] = v


def _ew(agg, u, relu=False, res=None):
    eb = 2000
    spec = pl.BlockSpec((1, eb, DS), lambda c, i: (c, i, 0))
    args = [agg, u]
    in_specs = [spec, spec]
    if res is not None:
        in_specs.append(spec)
        args.append(res)
    return pl.pallas_call(
        functools.partial(_ew_body, relu, res is not None),
        grid=(NSP, R // eb),
        in_specs=in_specs,
        out_specs=spec,
        out_shape=jax.ShapeDtypeStruct((NSP, R, DS), jnp.float32),
    )(*args)


DLAST = 16


def _mm_last_body(dual, *refs):
    parts = refs[:NSP]
    if dual:
        wl, ws, b, t_out, u_out = refs[NSP:]
    else:
        ws, b, u_out = refs[NSP:]
    a = _cat(parts)
    if dual:
        t_out[...] = jnp.dot(a, wl[...], preferred_element_type=jnp.float32)
    u_out[...] = jnp.dot(a, ws[...], preferred_element_type=jnp.float32) + b[...]


def _mm_last(h, wl, ws, b):
    dual = wl is not None
    in_specs = _part_specs()
    args = [h] * NSP
    out_spec = pl.BlockSpec((RB, DLAST), lambda i: (i, 0))
    if dual:
        in_specs.append(pl.BlockSpec((D, DLAST), lambda i: (0, 0)))
        args.append(wl)
    in_specs += [
        pl.BlockSpec((D, DLAST), lambda i: (0, 0)),
        pl.BlockSpec((1, DLAST), lambda i: (0, 0)),
    ]
    args += [ws, b.reshape(1, DLAST)]
    out_specs = [out_spec, out_spec] if dual else out_spec
    out_shape = jax.ShapeDtypeStruct((R, DLAST), jnp.float32)
    return pl.pallas_call(
        functools.partial(_mm_last_body, dual),
        grid=(R // RB,),
        in_specs=in_specs,
        out_specs=out_specs,
        out_shape=[out_shape, out_shape] if dual else out_shape,
    )(*args)


def _ew_last_body(agg, u, out):
    out[...] = agg[...] + u[...]


def _ew_last(agg, u):
    eb = 2000
    spec = pl.BlockSpec((eb, DLAST), lambda i: (i, 0))
    return pl.pallas_call(
        _ew_last_body,
        grid=(R // eb,),
        in_specs=[spec, spec],
        out_specs=spec,
        out_shape=jax.ShapeDtypeStruct((R, DLAST), jnp.float32),
    )(agg, u)


def kernel(vertices, local_features, global_features, params, edge_index):
    src = edge_index[0].reshape(NSUB, NCHUNK, CHUNK)
    dst = edge_index[1].reshape(NSUB, NCHUNK, CHUNK)
    n_dim = vertices.shape[2]
    d_loc = local_features.shape[2]


    def agg256(t):
        return _sc_agg(t.reshape(2 * R * 2, DSC), src, dst,
                       True, DSC).reshape(NSP, R, DS)

    p = params["block0"]
    t, u = _mm_first(vertices, local_features, global_features,
                     p["g1"]["Wl"], p["g1"]["Ws"], p["g1"]["b"], n_dim, d_loc)
    xs = None

    def bottleneck(xs, p, first_tu=None):
        if first_tu is not None:
            t, u = first_tu
            h1 = [_ew(agg256(t), u, relu=True)]
        else:
            t, u = _mm_dual(xs[0], p["g1"]["Wl"], p["g1"]["Ws"], p["g1"]["b"])
            h1 = [_ew(agg256(t), u, relu=True)]
            h1 += [_mm_single(x, p["g1"]["Ws"], p["g1"]["b"], relu=True)
                   for x in xs[1:]]
        t, u = _mm_dual(h1[0], p["g2"]["Wl"], p["g2"]["Ws"], p["g2"]["b"])
        h = [_ew(agg256(t), u, relu=True, res=h1[0])]
        h += [_mm_single(hj, p["g2"]["Ws"], p["g2"]["b"], relu=True, res=hj)
              for hj in h1[1:]]
        t, u = _mm_dual(h[0], p["g3"]["Wl"], p["g3"]["Ws"], p["g3"]["b"])
        x_out = [_ew(agg256(t), u)]
        x_out += [_mm_single(hj, p["g3"]["Ws"], p["g3"]["b"]) for hj in h[1:]]
        return x_out + h

    xs = bottleneck(None, params["block0"], first_tu=(t, u))
    xs = bottleneck(xs, params["block1"])
    xs = bottleneck(xs, params["block2"])

    pl_ = params["last"]
    wl16 = jnp.pad(pl_["Wl"], ((0, 0), (0, DLAST - 3)))
    ws16 = jnp.pad(pl_["Ws"], ((0, 0), (0, DLAST - 3)))
    b16 = jnp.pad(pl_["b"], (0, DLAST - 3))
    t, u0 = _mm_last(xs[0], wl16, ws16, b16)
    agg = _sc_agg(t, src, dst, False, DLAST)
    outs = [_ew_last(agg, u0)]
    outs += [_mm_last(x, None, ws16, b16) for x in xs[1:]]

    stacked = jnp.concatenate([o.reshape(B, N, DLAST) for o in outs], axis=1)
    return stacked[:, :, :3]


# speedup vs baseline: 54.0097x; 1.0535x over previous
import functools

import jax
import jax.numpy as jnp
from jax import lax
from jax.experimental import pallas as pl
from jax.experimental.pallas import tpu as pltpu
from jax.experimental.pallas import tpu_sc as plsc

N = 10000
B = 2
E = 160000
R = B * N
RB = 1000
D = 256
NSP = 2
DS = D // NSP
DSC = 64
UNITS = 4
NSUB = 16
NCORE = 2
EDGES_PER_SUB = E // NSUB
CHUNK = 80
NCHUNK = EDGES_PER_SUB // CHUNK
G = 5
NG = NCHUNK // G
RPS = N // NSUB


def _sc_agg_body(interleave, d, sup, src3, dst3, zeros_hbm, out,
                 dst_v, idx_v, rows_v, acc, sem, sem2):
    c = lax.axis_index("c")
    s = lax.axis_index("s")

    pltpu.sync_copy(src3.at[s], idx_v)
    pltpu.sync_copy(dst3.at[s], dst_v)

    r0 = s * RPS

    def adjust_idx(delta, double=False):
        def row(rr, carry):
            for j in range(CHUNK // 16):
                sl = pl.ds(j * 16, 16)
                v = idx_v[rr, sl]
                if double:
                    v = v + v
                idx_v[rr, sl] = v + delta
            return carry
        lax.fori_loop(0, NCHUNK, row, 0)

    if interleave:
        adjust_idx(c * (2 * R), double=True)
        deltas = [None, 2 * N, 1 - 2 * N, 2 * N]
        units = UNITS
    else:
        adjust_idx(c * N)
        units = 1

    for u in range(units):
        if u > 0:
            adjust_idx(deltas[u])

        pltpu.sync_copy(zeros_hbm, acc.at[pl.ds(r0, RPS)])
        plsc.subcore_barrier()

        def issue(g, bset, sm):
            for j in range(G):
                pltpu.async_copy(sup.at[idx_v.at[g * G + j]],
                                 rows_v.at[bset * G + j], sm)

        def drain_scatter(g, bset, sm):
            for j in range(G):
                pltpu.make_async_copy(sup.at[pl.ds(0, CHUNK)],
                                      rows_v.at[0], sm).wait()
            for j in range(G):
                pltpu.sync_copy(rows_v.at[bset * G + j],
                                acc.at[dst_v.at[g * G + j]], add=True)

        issue(0, 0, sem)

        def pair(p, carry):
            g0 = 2 * p
            issue(g0 + 1, 1, sem2)
            drain_scatter(g0, 0, sem)
            issue(g0 + 2, 0, sem)
            drain_scatter(g0 + 1, 1, sem2)
            return carry

        lax.fori_loop(0, (NG - 1) // 2, pair, 0)
        drain_scatter(NG - 1, 0, sem)
        plsc.subcore_barrier()

        if interleave:
            qq, b = u // 2, u % 2
            pltpu.sync_copy(acc.at[pl.ds(r0, RPS)],
                            out.at[pl.ds(c * R + b * N + r0, RPS), qq])
        else:
            pltpu.sync_copy(acc.at[pl.ds(r0, RPS)],
                            out.at[pl.ds(c * N + r0, RPS)])


@functools.partial(jax.jit, static_argnums=(3, 4))
def _sc_agg(sup, src3, dst3, interleave, d):
    mesh = plsc.VectorSubcoreMesh(core_axis_name="c", subcore_axis_name="s")
    body = functools.partial(_sc_agg_body, interleave, d)
    zeros_hbm = jnp.zeros((RPS, d), jnp.float32)
    out_shape = (2 * R, 2, DSC) if interleave else (R, d)
    return pl.kernel(
        body,
        out_type=jax.ShapeDtypeStruct(out_shape, jnp.float32),
        mesh=mesh,
        compiler_params=pltpu.CompilerParams(use_tc_tiling_on_sc=False),
        scratch_types=[
            pltpu.VMEM((NCHUNK, CHUNK), jnp.int32),
            pltpu.VMEM((NCHUNK, CHUNK), jnp.int32),
            pltpu.VMEM((2 * G, CHUNK, d), jnp.float32),
            pltpu.VMEM_SHARED((N, d), jnp.float32),
            pltpu.SemaphoreType.DMA,
            pltpu.SemaphoreType.DMA,
        ],
    )(sup, src3, dst3, zeros_hbm)


def _part_specs():
    def mk(q):
        return pl.BlockSpec((1, RB, DS), lambda i, q=q: (q, i, 0))
    return [mk(q) for q in range(NSP)]


def _cat(refs):
    return jnp.concatenate([r[0] for r in refs], axis=1)


def _split_store(out_ref, val):
    for q in range(NSP):
        out_ref[q] = val[:, q * DS:(q + 1) * DS]


_FULL_SPEC = pl.BlockSpec((NSP, RB, DS), lambda i: (0, i, 0))


def _mm_dual_body(*refs):
    parts, (wl, ws, b, t_out, u_out) = refs[:NSP], refs[NSP:]
    a = _cat(parts)
    _split_store(t_out, jnp.dot(a, wl[...], preferred_element_type=jnp.float32))
    _split_store(u_out, jnp.dot(a, ws[...], preferred_element_type=jnp.float32)
                 + b[...])


def _mm_dual(h, wl, ws, b):
    return pl.pallas_call(
        _mm_dual_body,
        grid=(R // RB,),
        in_specs=_part_specs() + [
            pl.BlockSpec((D, D), lambda i: (0, 0)),
            pl.BlockSpec((D, D), lambda i: (0, 0)),
            pl.BlockSpec((1, D), lambda i: (0, 0)),
        ],
        out_specs=[_FULL_SPEC, _FULL_SPEC],
        out_shape=[
            jax.ShapeDtypeStruct((NSP, R, DS), jnp.float32),
            jax.ShapeDtypeStruct((NSP, R, DS), jnp.float32),
        ],
    )(*([h] * NSP), wl, ws, b.reshape(1, D))


def _mm_single_body(relu, res, *refs):
    parts = refs[:NSP]
    if res:
        ws, b, res_ref, out = refs[NSP:]
    else:
        ws, b, out = refs[NSP:]
    a = _cat(parts)
    u = jnp.dot(a, ws[...], preferred_element_type=jnp.float32) + b[...]
    if relu:
        u = jnp.maximum(u, 0.0)
    if res:
        u = u + jnp.concatenate([res_ref[q] for q in range(NSP)], axis=1)
    _split_store(out, u)


def _mm_single(h, ws, b, relu=False, res=None):
    in_specs = _part_specs() + [
        pl.BlockSpec((D, D), lambda i: (0, 0)),
        pl.BlockSpec((1, D), lambda i: (0, 0)),
    ]
    args = [h] * NSP + [ws, b.reshape(1, D)]
    if res is not None:
        in_specs.append(_FULL_SPEC)
        args.append(res)
    return pl.pallas_call(
        functools.partial(_mm_single_body, relu, res is not None),
        grid=(R // RB,),
        in_specs=in_specs,
        out_specs=_FULL_SPEC,
        out_shape=jax.ShapeDtypeStruct((NSP, R, DS), jnp.float32),
    )(*args)


def _mm_first_body(blocks_per_batch, n_dim, av, af, g, wvl, wfl, wgl,
                   wvs, wfs, wgs, b, t_out, u_out):
    a3 = av[0]
    afull = af[0]
    gv = g[...]
    b_idx = pl.program_id(0) // blocks_per_batch

    def pick(m):
        return jnp.where(b_idx == 0, m[0:1, :], m[1:2, :])

    def vterm(w):
        acc = a3[:, 0:1] * w[0:1, :]
        for k in range(1, n_dim):
            acc += a3[:, k:k + 1] * w[k:k + 1, :]
        return acc

    gl = pick(jnp.dot(gv, wgl[...], preferred_element_type=jnp.float32))
    gs = pick(jnp.dot(gv, wgs[...], preferred_element_type=jnp.float32))
    t = (jnp.dot(afull, wfl[...], preferred_element_type=jnp.float32)
         + vterm(wvl) + gl)
    u = (jnp.dot(afull, wfs[...], preferred_element_type=jnp.float32)
         + vterm(wvs) + gs + b[...])
    _split_store(t_out, t)
    _split_store(u_out, u)


def _mm_first(verts, loc, glob, wl, ws, b, n_dim, d_loc):
    d_glob = glob.shape[1]
    wvl, wfl, wgl = wl[:n_dim], wl[n_dim:n_dim + d_loc], wl[n_dim + d_loc:]
    wvs, wfs, wgs = ws[:n_dim], ws[n_dim:n_dim + d_loc], ws[n_dim + d_loc:]
    bpb = N // RB

    def row3(i):
        return (i // bpb, i % bpb, 0)

    return pl.pallas_call(
        functools.partial(_mm_first_body, bpb, n_dim),
        grid=(R // RB,),
        in_specs=[
            pl.BlockSpec((1, RB, n_dim), row3),
            pl.BlockSpec((1, RB, d_loc), row3),
            pl.BlockSpec((B, d_glob), lambda i: (0, 0)),
            pl.BlockSpec((n_dim, D), lambda i: (0, 0)),
            pl.BlockSpec((d_loc, D), lambda i: (0, 0)),
            pl.BlockSpec((d_glob, D), lambda i: (0, 0)),
            pl.BlockSpec((n_dim, D), lambda i: (0, 0)),
            pl.BlockSpec((d_loc, D), lambda i: (0, 0)),
            pl.BlockSpec((d_glob, D), lambda i: (0, 0)),
            pl.BlockSpec((1, D), lambda i: (0, 0)),
        ],
        out_specs=[_FULL_SPEC, _FULL_SPEC],
        out_shape=[
            jax.ShapeDtypeStruct((NSP, R, DS), jnp.float32),
            jax.ShapeDtypeStruct((NSP, R, DS), jnp.float32),
        ],
    )(verts, loc, glob, wvl, wfl, wgl, wvs, wfs, wgs, b.reshape(1, D))


def _ew_body(relu, res, agg, u, *rest):
    if res:
        res_ref, out = rest
    else:
        (out,) = rest
    v = agg[...] + u[...]
    if relu:
        v = jnp.maximum(v, 0.0)
    if res:
        v = v + res_ref[...]
    out[...] = v


def _ew(agg, u, relu=False, res=None):
    eb = 2000
    spec = pl.BlockSpec((1, eb, DS), lambda c, i: (c, i, 0))
    args = [agg, u]
    in_specs = [spec, spec]
    if res is not None:
        in_specs.append(spec)
        args.append(res)
    return pl.pallas_call(
        functools.partial(_ew_body, relu, res is not None),
        grid=(NSP, R // eb),
        in_specs=in_specs,
        out_specs=spec,
        out_shape=jax.ShapeDtypeStruct((NSP, R, DS), jnp.float32),
    )(*args)


DLAST = 16


def _mm_last_body(dual, *refs):
    parts = refs[:NSP]
    if dual:
        wl, ws, b, t_out, u_out = refs[NSP:]
    else:
        ws, b, u_out = refs[NSP:]
    a = _cat(parts)
    if dual:
        t_out[...] = jnp.dot(a, wl[...], preferred_element_type=jnp.float32)
    u_out[...] = jnp.dot(a, ws[...], preferred_element_type=jnp.float32) + b[...]


def _mm_last(h, wl, ws, b):
    dual = wl is not None
    in_specs = _part_specs()
    args = [h] * NSP
    out_spec = pl.BlockSpec((RB, DLAST), lambda i: (i, 0))
    if dual:
        in_specs.append(pl.BlockSpec((D, DLAST), lambda i: (0, 0)))
        args.append(wl)
    in_specs += [
        pl.BlockSpec((D, DLAST), lambda i: (0, 0)),
        pl.BlockSpec((1, DLAST), lambda i: (0, 0)),
    ]
    args += [ws, b.reshape(1, DLAST)]
    out_specs = [out_spec, out_spec] if dual else out_spec
    out_shape = jax.ShapeDtypeStruct((R, DLAST), jnp.float32)
    return pl.pallas_call(
        functools.partial(_mm_last_body, dual),
        grid=(R // RB,),
        in_specs=in_specs,
        out_specs=out_specs,
        out_shape=[out_shape, out_shape] if dual else out_shape,
    )(*args)


def _ew_last_body(agg, u, out):
    out[...] = agg[...] + u[...]


def _ew_last(agg, u):
    eb = 2000
    spec = pl.BlockSpec((eb, DLAST), lambda i: (i, 0))
    return pl.pallas_call(
        _ew_last_body,
        grid=(R // eb,),
        in_specs=[spec, spec],
        out_specs=spec,
        out_shape=jax.ShapeDtypeStruct((R, DLAST), jnp.float32),
    )(agg, u)


def kernel(vertices, local_features, global_features, params, edge_index):
    src = edge_index[0].reshape(NSUB, NCHUNK, CHUNK)
    dst = edge_index[1].reshape(NSUB, NCHUNK, CHUNK)
    n_dim = vertices.shape[2]
    d_loc = local_features.shape[2]


    def agg256(t):
        return _sc_agg(t.reshape(2 * R * 2, DSC), src, dst,
                       True, DSC).reshape(NSP, R, DS)

    p = params["block0"]
    t, u = _mm_first(vertices, local_features, global_features,
                     p["g1"]["Wl"], p["g1"]["Ws"], p["g1"]["b"], n_dim, d_loc)
    xs = None

    def bottleneck(xs, p, first_tu=None):
        if first_tu is not None:
            t, u = first_tu
            h1 = [_ew(agg256(t), u, relu=True)]
        else:
            t, u = _mm_dual(xs[0], p["g1"]["Wl"], p["g1"]["Ws"], p["g1"]["b"])
            h1 = [_ew(agg256(t), u, relu=True)]
            h1 += [_mm_single(x, p["g1"]["Ws"], p["g1"]["b"], relu=True)
                   for x in xs[1:]]
        t, u = _mm_dual(h1[0], p["g2"]["Wl"], p["g2"]["Ws"], p["g2"]["b"])
        h = [_ew(agg256(t), u, relu=True, res=h1[0])]
        h += [_mm_single(hj, p["g2"]["Ws"], p["g2"]["b"], relu=True, res=hj)
              for hj in h1[1:]]
        t, u = _mm_dual(h[0], p["g3"]["Wl"], p["g3"]["Ws"], p["g3"]["b"])
        x_out = [_ew(agg256(t), u)]
        x_out += [_mm_single(hj, p["g3"]["Ws"], p["g3"]["b"]) for hj in h[1:]]
        return x_out + h

    xs = bottleneck(None, params["block0"], first_tu=(t, u))
    xs = bottleneck(xs, params["block1"])
    xs = bottleneck(xs, params["block2"])

    pl_ = params["last"]
    wl16 = jnp.pad(pl_["Wl"], ((0, 0), (0, DLAST - 3)))
    ws16 = jnp.pad(pl_["Ws"], ((0, 0), (0, DLAST - 3)))
    b16 = jnp.pad(pl_["b"], (0, DLAST - 3))
    t, u0 = _mm_last(xs[0], wl16, ws16, b16)
    agg = _sc_agg(t, src, dst, False, DLAST)
    outs = [_ew_last(agg, u0)]
    outs += [_mm_last(x, None, ws16, b16) for x in xs[1:]]

    stacked = jnp.concatenate([o.reshape(B, N, DLAST) for o in outs], axis=1)
    return stacked[:, :, :3]


# speedup vs baseline: 56.5917x; 1.0478x over previous
import functools

import jax
import jax.numpy as jnp
from jax import lax
from jax.experimental import pallas as pl
from jax.experimental.pallas import tpu as pltpu
from jax.experimental.pallas import tpu_sc as plsc

N = 10000
B = 2
E = 160000
R = B * N
RB = 1000
D = 256
NSP = 2
DS = D // NSP
DSC = 64
UNITS = 4
NSUB = 16
NCORE = 2
EDGES_PER_SUB = E // NSUB
CHUNK = 80
NCHUNK = EDGES_PER_SUB // CHUNK
G = 5
NG = NCHUNK // G
RPS = N // NSUB


def _sc_agg_body(interleave, d, sup, src3, dst3, zeros_hbm, out,
                 dst_v, idx_v, rows_v, acc, sem, sem2):
    c = lax.axis_index("c")
    s = lax.axis_index("s")

    pltpu.sync_copy(src3.at[s], idx_v)
    pltpu.sync_copy(dst3.at[s], dst_v)

    r0 = s * RPS

    def adjust_idx(delta, double=False):
        def row(rr, carry):
            for j in range(CHUNK // 16):
                sl = pl.ds(j * 16, 16)
                v = idx_v[rr, sl]
                if double:
                    v = v + v
                idx_v[rr, sl] = v + delta
            return carry
        lax.fori_loop(0, NCHUNK, row, 0)

    if interleave:
        adjust_idx(c * (2 * R), double=True)
        deltas = [None, 2 * N, 1 - 2 * N, 2 * N]
        units = UNITS
    else:
        adjust_idx(c * N)
        units = 1

    for u in range(units):
        if u > 0:
            adjust_idx(deltas[u])

        pltpu.sync_copy(zeros_hbm, acc.at[pl.ds(r0, RPS)])
        plsc.subcore_barrier()

        def issue(g, bset, sm):
            for j in range(G):
                pltpu.async_copy(sup.at[idx_v.at[g * G + j]],
                                 rows_v.at[bset * G + j], sm)

        def drain_scatter(g, bset, sm):
            for j in range(G):
                pltpu.make_async_copy(sup.at[pl.ds(0, CHUNK)],
                                      rows_v.at[0], sm).wait()
            for j in range(G):
                pltpu.sync_copy(rows_v.at[bset * G + j],
                                acc.at[dst_v.at[g * G + j]], add=True)

        issue(0, 0, sem)

        def pair(p, carry):
            g0 = 2 * p
            issue(g0 + 1, 1, sem2)
            drain_scatter(g0, 0, sem)
            issue(g0 + 2, 0, sem)
            drain_scatter(g0 + 1, 1, sem2)
            return carry

        lax.fori_loop(0, (NG - 1) // 2, pair, 0)
        drain_scatter(NG - 1, 0, sem)
        plsc.subcore_barrier()

        if interleave:
            qq, b = u // 2, u % 2
            pltpu.sync_copy(acc.at[pl.ds(r0, RPS)],
                            out.at[pl.ds(c * R + b * N + r0, RPS), qq])
        else:
            pltpu.sync_copy(acc.at[pl.ds(r0, RPS)],
                            out.at[pl.ds(c * N + r0, RPS)])


@functools.partial(jax.jit, static_argnums=(3, 4))
def _sc_agg(sup, src3, dst3, interleave, d):
    mesh = plsc.VectorSubcoreMesh(core_axis_name="c", subcore_axis_name="s")
    body = functools.partial(_sc_agg_body, interleave, d)
    zeros_hbm = jnp.zeros((RPS, d), jnp.float32)
    out_shape = (2 * R, 2, DSC) if interleave else (R, d)
    return pl.kernel(
        body,
        out_type=jax.ShapeDtypeStruct(out_shape, jnp.float32),
        mesh=mesh,
        compiler_params=pltpu.CompilerParams(use_tc_tiling_on_sc=False),
        scratch_types=[
            pltpu.VMEM((NCHUNK, CHUNK), jnp.int32),
            pltpu.VMEM((NCHUNK, CHUNK), jnp.int32),
            pltpu.VMEM((2 * G, CHUNK, d), jnp.float32),
            pltpu.VMEM_SHARED((N, d), jnp.float32),
            pltpu.SemaphoreType.DMA,
            pltpu.SemaphoreType.DMA,
        ],
    )(sup, src3, dst3, zeros_hbm)


def _part_specs():
    def mk(q):
        return pl.BlockSpec((1, RB, DS), lambda i, q=q: (q, i, 0))
    return [mk(q) for q in range(NSP)]


def _cat(refs):
    return jnp.concatenate([r[0] for r in refs], axis=1)


def _split_store(out_ref, val):
    for q in range(NSP):
        out_ref[q] = val[:, q * DS:(q + 1) * DS]


_FULL_SPEC = pl.BlockSpec((NSP, RB, DS), lambda i: (0, i, 0))


def _mm_dual_body(*refs):
    parts, (wl, ws, b, t_out, u_out) = refs[:NSP], refs[NSP:]
    a = _cat(parts)
    _split_store(t_out, jnp.dot(a, wl[...], preferred_element_type=jnp.float32))
    _split_store(u_out, jnp.dot(a, ws[...], preferred_element_type=jnp.float32)
                 + b[...])


def _mm_dual(h, wl, ws, b):
    return pl.pallas_call(
        _mm_dual_body,
        grid=(R // RB,),
        in_specs=_part_specs() + [
            pl.BlockSpec((D, D), lambda i: (0, 0)),
            pl.BlockSpec((D, D), lambda i: (0, 0)),
            pl.BlockSpec((1, D), lambda i: (0, 0)),
        ],
        out_specs=[_FULL_SPEC, _FULL_SPEC],
        out_shape=[
            jax.ShapeDtypeStruct((NSP, R, DS), jnp.float32),
            jax.ShapeDtypeStruct((NSP, R, DS), jnp.float32),
        ],
    )(*([h] * NSP), wl, ws, b.reshape(1, D))


def _combine(relu, has_res, agg, uprev, res_ref):
    xv = agg[...] + uprev[...]
    if relu:
        xv = jnp.maximum(xv, 0.0)
    if has_res:
        xv = xv + res_ref[...]
    return xv


def _mm_dual_fused_body(relu, has_res, want_x, *refs):
    if has_res:
        agg, uprev, res_ref, wl, ws, b, *outs = refs
    else:
        agg, uprev, wl, ws, b, *outs = refs
        res_ref = None
    xv = _combine(relu, has_res, agg, uprev, res_ref)
    if want_x:
        x_out, t_out, u_out = outs
        x_out[...] = xv
    else:
        t_out, u_out = outs
    a = jnp.concatenate([xv[q] for q in range(NSP)], axis=-1)
    _split_store(t_out, jnp.dot(a, wl[...], preferred_element_type=jnp.float32))
    _split_store(u_out, jnp.dot(a, ws[...], preferred_element_type=jnp.float32)
                 + b[...])


def _mm_dual_fused(agg, uprev, wl, ws, b, relu=False, res=None, want_x=True):
    in_specs = [_FULL_SPEC, _FULL_SPEC]
    args = [agg, uprev]
    if res is not None:
        in_specs.append(_FULL_SPEC)
        args.append(res)
    in_specs += [
        pl.BlockSpec((D, D), lambda i: (0, 0)),
        pl.BlockSpec((D, D), lambda i: (0, 0)),
        pl.BlockSpec((1, D), lambda i: (0, 0)),
    ]
    args += [wl, ws, b.reshape(1, D)]
    shp = jax.ShapeDtypeStruct((NSP, R, DS), jnp.float32)
    n_out = 3 if want_x else 2
    return pl.pallas_call(
        functools.partial(_mm_dual_fused_body, relu, res is not None, want_x),
        grid=(R // RB,),
        in_specs=in_specs,
        out_specs=[_FULL_SPEC] * n_out,
        out_shape=[shp] * n_out,
    )(*args)


def _mm_last_fused_body(agg, uprev, wl, ws, b, t_out, u_out):
    xv = _combine(False, False, agg, uprev, None)
    a = jnp.concatenate([xv[q] for q in range(NSP)], axis=-1)
    t_out[...] = jnp.dot(a, wl[...], preferred_element_type=jnp.float32)
    u_out[...] = jnp.dot(a, ws[...], preferred_element_type=jnp.float32) + b[...]


def _mm_last_fused(agg, uprev, wl, ws, b):
    shp16 = jax.ShapeDtypeStruct((R, DLAST), jnp.float32)
    out_spec16 = pl.BlockSpec((RB, DLAST), lambda i: (i, 0))
    return pl.pallas_call(
        _mm_last_fused_body,
        grid=(R // RB,),
        in_specs=[
            _FULL_SPEC, _FULL_SPEC,
            pl.BlockSpec((D, DLAST), lambda i: (0, 0)),
            pl.BlockSpec((D, DLAST), lambda i: (0, 0)),
            pl.BlockSpec((1, DLAST), lambda i: (0, 0)),
        ],
        out_specs=[out_spec16, out_spec16],
        out_shape=[shp16, shp16],
    )(agg, uprev, wl, ws, b.reshape(1, DLAST))


def _mm_single_body(relu, res, *refs):
    parts = refs[:NSP]
    if res:
        ws, b, res_ref, out = refs[NSP:]
    else:
        ws, b, out = refs[NSP:]
    a = _cat(parts)
    u = jnp.dot(a, ws[...], preferred_element_type=jnp.float32) + b[...]
    if relu:
        u = jnp.maximum(u, 0.0)
    if res:
        u = u + jnp.concatenate([res_ref[q] for q in range(NSP)], axis=1)
    _split_store(out, u)


def _mm_single(h, ws, b, relu=False, res=None):
    in_specs = _part_specs() + [
        pl.BlockSpec((D, D), lambda i: (0, 0)),
        pl.BlockSpec((1, D), lambda i: (0, 0)),
    ]
    args = [h] * NSP + [ws, b.reshape(1, D)]
    if res is not None:
        in_specs.append(_FULL_SPEC)
        args.append(res)
    return pl.pallas_call(
        functools.partial(_mm_single_body, relu, res is not None),
        grid=(R // RB,),
        in_specs=in_specs,
        out_specs=_FULL_SPEC,
        out_shape=jax.ShapeDtypeStruct((NSP, R, DS), jnp.float32),
    )(*args)


def _mm_first_body(blocks_per_batch, n_dim, av, af, g, wvl, wfl, wgl,
                   wvs, wfs, wgs, b, t_out, u_out):
    a3 = av[0]
    afull = af[0]
    gv = g[...]
    b_idx = pl.program_id(0) // blocks_per_batch

    def pick(m):
        return jnp.where(b_idx == 0, m[0:1, :], m[1:2, :])

    def vterm(w):
        acc = a3[:, 0:1] * w[0:1, :]
        for k in range(1, n_dim):
            acc += a3[:, k:k + 1] * w[k:k + 1, :]
        return acc

    gl = pick(jnp.dot(gv, wgl[...], preferred_element_type=jnp.float32))
    gs = pick(jnp.dot(gv, wgs[...], preferred_element_type=jnp.float32))
    t = (jnp.dot(afull, wfl[...], preferred_element_type=jnp.float32)
         + vterm(wvl) + gl)
    u = (jnp.dot(afull, wfs[...], preferred_element_type=jnp.float32)
         + vterm(wvs) + gs + b[...])
    _split_store(t_out, t)
    _split_store(u_out, u)


def _mm_first(verts, loc, glob, wl, ws, b, n_dim, d_loc):
    d_glob = glob.shape[1]
    wvl, wfl, wgl = wl[:n_dim], wl[n_dim:n_dim + d_loc], wl[n_dim + d_loc:]
    wvs, wfs, wgs = ws[:n_dim], ws[n_dim:n_dim + d_loc], ws[n_dim + d_loc:]
    bpb = N // RB

    def row3(i):
        return (i // bpb, i % bpb, 0)

    return pl.pallas_call(
        functools.partial(_mm_first_body, bpb, n_dim),
        grid=(R // RB,),
        in_specs=[
            pl.BlockSpec((1, RB, n_dim), row3),
            pl.BlockSpec((1, RB, d_loc), row3),
            pl.BlockSpec((B, d_glob), lambda i: (0, 0)),
            pl.BlockSpec((n_dim, D), lambda i: (0, 0)),
            pl.BlockSpec((d_loc, D), lambda i: (0, 0)),
            pl.BlockSpec((d_glob, D), lambda i: (0, 0)),
            pl.BlockSpec((n_dim, D), lambda i: (0, 0)),
            pl.BlockSpec((d_loc, D), lambda i: (0, 0)),
            pl.BlockSpec((d_glob, D), lambda i: (0, 0)),
            pl.BlockSpec((1, D), lambda i: (0, 0)),
        ],
        out_specs=[_FULL_SPEC, _FULL_SPEC],
        out_shape=[
            jax.ShapeDtypeStruct((NSP, R, DS), jnp.float32),
            jax.ShapeDtypeStruct((NSP, R, DS), jnp.float32),
        ],
    )(verts, loc, glob, wvl, wfl, wgl, wvs, wfs, wgs, b.reshape(1, D))


def _ew_body(relu, res, agg, u, *rest):
    if res:
        res_ref, out = rest
    else:
        (out,) = rest
    v = agg[...] + u[...]
    if relu:
        v = jnp.maximum(v, 0.0)
    if res:
        v = v + res_ref[...]
    out[...] = v


def _ew(agg, u, relu=False, res=None):
    eb = 2000
    spec = pl.BlockSpec((1, eb, DS), lambda c, i: (c, i, 0))
    args = [agg, u]
    in_specs = [spec, spec]
    if res is not None:
        in_specs.append(spec)
        args.append(res)
    return pl.pallas_call(
        functools.partial(_ew_body, relu, res is not None),
        grid=(NSP, R // eb),
        in_specs=in_specs,
        out_specs=spec,
        out_shape=jax.ShapeDtypeStruct((NSP, R, DS), jnp.float32),
    )(*args)


DLAST = 16


def _mm_last_body(dual, *refs):
    parts = refs[:NSP]
    if dual:
        wl, ws, b, t_out, u_out = refs[NSP:]
    else:
        ws, b, u_out = refs[NSP:]
    a = _cat(parts)
    if dual:
        t_out[...] = jnp.dot(a, wl[...], preferred_element_type=jnp.float32)
    u_out[...] = jnp.dot(a, ws[...], preferred_element_type=jnp.float32) + b[...]


def _mm_last(h, wl, ws, b):
    dual = wl is not None
    in_specs = _part_specs()
    args = [h] * NSP
    out_spec = pl.BlockSpec((RB, DLAST), lambda i: (i, 0))
    if dual:
        in_specs.append(pl.BlockSpec((D, DLAST), lambda i: (0, 0)))
        args.append(wl)
    in_specs += [
        pl.BlockSpec((D, DLAST), lambda i: (0, 0)),
        pl.BlockSpec((1, DLAST), lambda i: (0, 0)),
    ]
    args += [ws, b.reshape(1, DLAST)]
    out_specs = [out_spec, out_spec] if dual else out_spec
    out_shape = jax.ShapeDtypeStruct((R, DLAST), jnp.float32)
    return pl.pallas_call(
        functools.partial(_mm_last_body, dual),
        grid=(R // RB,),
        in_specs=in_specs,
        out_specs=out_specs,
        out_shape=[out_shape, out_shape] if dual else out_shape,
    )(*args)


def _ew_last_body(agg, u, out):
    out[...] = agg[...] + u[...]


def _ew_last(agg, u):
    eb = 2000
    spec = pl.BlockSpec((eb, DLAST), lambda i: (i, 0))
    return pl.pallas_call(
        _ew_last_body,
        grid=(R // eb,),
        in_specs=[spec, spec],
        out_specs=spec,
        out_shape=jax.ShapeDtypeStruct((R, DLAST), jnp.float32),
    )(agg, u)


def kernel(vertices, local_features, global_features, params, edge_index):
    src = edge_index[0].reshape(NSUB, NCHUNK, CHUNK)
    dst = edge_index[1].reshape(NSUB, NCHUNK, CHUNK)
    n_dim = vertices.shape[2]
    d_loc = local_features.shape[2]


    def agg256(t):
        return _sc_agg(t.reshape(2 * R * 2, DSC), src, dst,
                       True, DSC).reshape(NSP, R, DS)

    p0 = params["block0"]
    t, u = _mm_first(vertices, local_features, global_features,
                     p0["g1"]["Wl"], p0["g1"]["Ws"], p0["g1"]["b"],
                     n_dim, d_loc)

    def bottleneck(xs, pend, p):
        if pend is None:
            t1, u1 = t, u
            h1_rest = []
        else:
            t1, u1 = _mm_dual_fused(pend[0], pend[1], p["g1"]["Wl"],
                                    p["g1"]["Ws"], p["g1"]["b"], want_x=False)
            h1_rest = [_mm_single(x, p["g1"]["Ws"], p["g1"]["b"], relu=True)
                       for x in xs[1:]]
        agg1 = agg256(t1)
        h1_0, t2, u2 = _mm_dual_fused(agg1, u1, p["g2"]["Wl"], p["g2"]["Ws"],
                                      p["g2"]["b"], relu=True)
        agg2 = agg256(t2)
        h_0, t3, u3 = _mm_dual_fused(agg2, u2, p["g3"]["Wl"], p["g3"]["Ws"],
                                     p["g3"]["b"], relu=True, res=h1_0)
        agg3 = agg256(t3)
        h1 = [h1_0] + h1_rest
        h = [h_0] + [_mm_single(hj, p["g2"]["Ws"], p["g2"]["b"],
                                relu=True, res=hj) for hj in h1_rest]
        x_out = [None]
        x_out += [_mm_single(hj, p["g3"]["Ws"], p["g3"]["b"]) for hj in h[1:]]
        return x_out + h, (agg3, u3)

    xs, pend = bottleneck(None, None, params["block0"])
    xs, pend = bottleneck(xs, pend, params["block1"])
    xs, pend = bottleneck(xs, pend, params["block2"])

    pl_ = params["last"]
    wl16 = jnp.pad(pl_["Wl"], ((0, 0), (0, DLAST - 3)))
    ws16 = jnp.pad(pl_["Ws"], ((0, 0), (0, DLAST - 3)))
    b16 = jnp.pad(pl_["b"], (0, DLAST - 3))
    t, u0 = _mm_last_fused(pend[0], pend[1], wl16, ws16, b16)
    agg = _sc_agg(t, src, dst, False, DLAST)
    outs = [_ew_last(agg, u0)]
    outs += [_mm_last(x, None, ws16, b16) for x in xs[1:]]

    stacked = jnp.concatenate([o.reshape(B, N, DLAST) for o in outs], axis=1)
    return stacked[:, :, :3]
